# initial kernel scaffold (unmeasured)
import jax
import jax.numpy as jnp
from jax import lax
from jax.experimental import pallas as pl
from jax.experimental.pallas import tpu as pltpu

N_DEV = 32
M, N = 4096, 8192
CHUNK_M = M // N_DEV


def _allreduce_body(in_ref, out_ref, bufA, bufB, bufC,
                    send_rs, recv_rs, send_ag, recv_ag, copy_sem,
                    ack_rs, ack_ag):
    me = lax.axis_index("i")
    right = (me + 1) % N_DEV
    left = (me + N_DEV - 1) % N_DEV

    barrier_sem = pltpu.get_barrier_semaphore()
    for nbr in (left, right):
        pl.semaphore_signal(barrier_sem, inc=1, device_id=(nbr,),
                            device_id_type=pl.DeviceIdType.MESH)
    pl.semaphore_wait(barrier_sem, 2)

    cp = pltpu.make_async_copy(
        in_ref.at[pl.ds(me * CHUNK_M, CHUNK_M)], bufA, copy_sem)
    cp.start()
    cp.wait()

    for s in range(N_DEV - 1):
        rdma = pltpu.make_async_remote_copy(
            src_ref=bufA, dst_ref=bufB,
            send_sem=send_rs, recv_sem=recv_rs,
            device_id=(right,), device_id_type=pl.DeviceIdType.MESH)
        rdma.start()
        idx = (me + (N_DEV - 1 - s)) % N_DEV
        cp = pltpu.make_async_copy(
            in_ref.at[pl.ds(idx * CHUNK_M, CHUNK_M)], bufC, copy_sem)
        cp.start()
        rdma.wait()
        cp.wait()
        bufA[...] = bufB[...] + bufC[...]
        pl.semaphore_signal(ack_rs, inc=1, device_id=(left,),
                            device_id_type=pl.DeviceIdType.MESH)
        pl.semaphore_wait(ack_rs, 1)

    rc = (me + 1) % N_DEV
    cp = pltpu.make_async_copy(
        bufA, out_ref.at[pl.ds(rc * CHUNK_M, CHUNK_M)], copy_sem)
    cp.start()
    cp.wait()

    for s in range(N_DEV - 1):
        idx = (me + (N_DEV + 1 - s)) % N_DEV
        sl = pl.ds(idx * CHUNK_M, CHUNK_M)
        rdma = pltpu.make_async_remote_copy(
            src_ref=out_ref.at[sl], dst_ref=out_ref.at[sl],
            send_sem=send_ag, recv_sem=recv_ag,
            device_id=(right,), device_id_type=pl.DeviceIdType.MESH)
        rdma.start()
        rdma.wait()
        pl.semaphore_signal(ack_ag, inc=1, device_id=(left,),
                            device_id_type=pl.DeviceIdType.MESH)
        pl.semaphore_wait(ack_ag, 1)


def _ring_allreduce(partial):
    return pl.pallas_call(
        _allreduce_body,
        out_shape=jax.ShapeDtypeStruct((M, N), jnp.float32),
        in_specs=[pl.BlockSpec(memory_space=pltpu.ANY)],
        out_specs=pl.BlockSpec(memory_space=pltpu.ANY),
        scratch_shapes=[
            pltpu.VMEM((CHUNK_M, N), jnp.float32),
            pltpu.VMEM((CHUNK_M, N), jnp.float32),
            pltpu.VMEM((CHUNK_M, N), jnp.float32),
            pltpu.SemaphoreType.DMA,
            pltpu.SemaphoreType.DMA,
            pltpu.SemaphoreType.DMA,
            pltpu.SemaphoreType.DMA,
            pltpu.SemaphoreType.DMA,
            pltpu.SemaphoreType.REGULAR,
            pltpu.SemaphoreType.REGULAR,
        ],
        compiler_params=pltpu.CompilerParams(collective_id=0),
    )(partial)


def kernel(x, w_mat, scale_x, scale_w):
    partial = jnp.dot(x.astype(jnp.bfloat16), w_mat.astype(jnp.bfloat16),
                      preferred_element_type=jnp.float32)
    partial = partial * (scale_x[0] * scale_w[0])
    return _ring_allreduce(partial)


# baseline (device time: 3391688 ns/iter reference)
import jax
import jax.numpy as jnp
from jax import lax
from jax.experimental import pallas as pl
from jax.experimental.pallas import tpu as pltpu

N_DEV = 32
M, N = 4096, 8192
CHUNK_M = M // N_DEV


def _allreduce_body(in_ref, out_ref, bufA, bufB, bufC,
                    send_rs, recv_rs, send_ag, recv_ag, copy_sem,
                    ack_rs, ack_ag):
    me = lax.axis_index("i")
    right = (me + 1) % N_DEV
    left = (me + N_DEV - 1) % N_DEV

    barrier_sem = pltpu.get_barrier_semaphore()
    for nbr in (left, right):
        pl.semaphore_signal(barrier_sem, inc=1, device_id=(nbr,),
                            device_id_type=pl.DeviceIdType.MESH)
    pl.semaphore_wait(barrier_sem, 2)

    cp = pltpu.make_async_copy(
        in_ref.at[pl.ds(me * CHUNK_M, CHUNK_M)], bufA, copy_sem)
    cp.start()
    cp.wait()

    for s in range(N_DEV - 1):
        rdma = pltpu.make_async_remote_copy(
            src_ref=bufA, dst_ref=bufB,
            send_sem=send_rs, recv_sem=recv_rs,
            device_id=(right,), device_id_type=pl.DeviceIdType.MESH)
        rdma.start()
        idx = (me + (N_DEV - 1 - s)) % N_DEV
        cp = pltpu.make_async_copy(
            in_ref.at[pl.ds(idx * CHUNK_M, CHUNK_M)], bufC, copy_sem)
        cp.start()
        rdma.wait()
        cp.wait()
        bufA[...] = bufB[...] + bufC[...]
        pl.semaphore_signal(ack_rs, inc=1, device_id=(left,),
                            device_id_type=pl.DeviceIdType.MESH)
        pl.semaphore_wait(ack_rs, 1)

    rc = (me + 1) % N_DEV
    cp = pltpu.make_async_copy(
        bufA, out_ref.at[pl.ds(rc * CHUNK_M, CHUNK_M)], copy_sem)
    cp.start()
    cp.wait()

    for s in range(N_DEV - 1):
        idx = (me + (N_DEV + 1 - s)) % N_DEV
        sl = pl.ds(idx * CHUNK_M, CHUNK_M)
        rdma = pltpu.make_async_remote_copy(
            src_ref=out_ref.at[sl], dst_ref=out_ref.at[sl],
            send_sem=send_ag, recv_sem=recv_ag,
            device_id=(right,), device_id_type=pl.DeviceIdType.MESH)
        rdma.start()
        rdma.wait()
        pl.semaphore_signal(ack_ag, inc=1, device_id=(left,),
                            device_id_type=pl.DeviceIdType.MESH)
        pl.semaphore_wait(ack_ag, 1)


def _ring_allreduce(partial):
    return pl.pallas_call(
        _allreduce_body,
        out_shape=jax.ShapeDtypeStruct((M, N), jnp.float32),
        in_specs=[pl.BlockSpec(memory_space=pl.ANY)],
        out_specs=pl.BlockSpec(memory_space=pl.ANY),
        scratch_shapes=[
            pltpu.VMEM((CHUNK_M, N), jnp.float32),
            pltpu.VMEM((CHUNK_M, N), jnp.float32),
            pltpu.VMEM((CHUNK_M, N), jnp.float32),
            pltpu.SemaphoreType.DMA,
            pltpu.SemaphoreType.DMA,
            pltpu.SemaphoreType.DMA,
            pltpu.SemaphoreType.DMA,
            pltpu.SemaphoreType.DMA,
            pltpu.SemaphoreType.REGULAR,
            pltpu.SemaphoreType.REGULAR,
        ],
        compiler_params=pltpu.CompilerParams(collective_id=0),
    )(partial)


def kernel(x, w_mat, scale_x, scale_w):
    partial = jnp.dot(x.astype(jnp.bfloat16), w_mat.astype(jnp.bfloat16),
                      preferred_element_type=jnp.float32)
    partial = partial * (scale_x[0] * scale_w[0])
    return _ring_allreduce(partial)


# device time: 3099017 ns/iter; 1.0944x vs baseline; 1.0944x over previous
import jax
import jax.numpy as jnp
from jax import lax
from jax.experimental import pallas as pl
from jax.experimental.pallas import tpu as pltpu

N_DEV = 32
M, N = 4096, 8192
CHUNK_M = M // N_DEV

LANE_DIRS = (1, -1)
NLANES = len(LANE_DIRS)
LANE_W = N // NLANES


def _allreduce_body(in_ref, out_ref, bufA, bufB, bufC,
                    send_rs, recv_rs, send_ag, recv_ag, copy_sem,
                    ack_rs, ack_ag):
    me = lax.axis_index("i")
    right = (me + 1) % N_DEV
    left = (me + N_DEV - 1) % N_DEV
    down = [(me + d) % N_DEV for d in LANE_DIRS]
    up = [(me - d) % N_DEV for d in LANE_DIRS]
    offs = [li * LANE_W for li in range(NLANES)]

    barrier_sem = pltpu.get_barrier_semaphore()
    for nbr in (left, right):
        pl.semaphore_signal(barrier_sem, inc=1, device_id=(nbr,),
                            device_id_type=pl.DeviceIdType.MESH)
    pl.semaphore_wait(barrier_sem, 2)

    for li in range(NLANES):
        pltpu.make_async_copy(
            in_ref.at[pl.ds(me * CHUNK_M, CHUNK_M), pl.ds(offs[li], LANE_W)],
            bufA.at[li], copy_sem.at[li]).start()
    for li in range(NLANES):
        pltpu.make_async_copy(
            in_ref.at[pl.ds(me * CHUNK_M, CHUNK_M), pl.ds(offs[li], LANE_W)],
            bufA.at[li], copy_sem.at[li]).wait()

    for s in range(N_DEV - 1):
        rdmas = []
        for li, d in enumerate(LANE_DIRS):
            rdma = pltpu.make_async_remote_copy(
                src_ref=bufA.at[li], dst_ref=bufB.at[li],
                send_sem=send_rs.at[li], recv_sem=recv_rs.at[li],
                device_id=(down[li],), device_id_type=pl.DeviceIdType.MESH)
            rdma.start()
            rdmas.append(rdma)
        cps = []
        for li, d in enumerate(LANE_DIRS):
            idx = (me - d * (s + 1)) % N_DEV
            cp = pltpu.make_async_copy(
                in_ref.at[pl.ds(idx * CHUNK_M, CHUNK_M),
                          pl.ds(offs[li], LANE_W)],
                bufC.at[li], copy_sem.at[li])
            cp.start()
            cps.append(cp)
        for li in range(NLANES):
            rdmas[li].wait()
            cps[li].wait()
            bufA[li] = bufB[li] + bufC[li]
        for li in range(NLANES):
            pl.semaphore_signal(ack_rs.at[li], inc=1, device_id=(up[li],),
                                device_id_type=pl.DeviceIdType.MESH)
        for li in range(NLANES):
            pl.semaphore_wait(ack_rs.at[li], 1)

    for li, d in enumerate(LANE_DIRS):
        rc = (me + d) % N_DEV
        pltpu.make_async_copy(
            bufA.at[li],
            out_ref.at[pl.ds(rc * CHUNK_M, CHUNK_M), pl.ds(offs[li], LANE_W)],
            copy_sem.at[li]).start()
    for li, d in enumerate(LANE_DIRS):
        rc = (me + d) % N_DEV
        pltpu.make_async_copy(
            bufA.at[li],
            out_ref.at[pl.ds(rc * CHUNK_M, CHUNK_M), pl.ds(offs[li], LANE_W)],
            copy_sem.at[li]).wait()

    for s in range(N_DEV - 1):
        rdmas = []
        for li, d in enumerate(LANE_DIRS):
            idx = (me + d * (1 - s)) % N_DEV
            sl = (pl.ds(idx * CHUNK_M, CHUNK_M), pl.ds(offs[li], LANE_W))
            rdma = pltpu.make_async_remote_copy(
                src_ref=out_ref.at[sl], dst_ref=out_ref.at[sl],
                send_sem=send_ag.at[li], recv_sem=recv_ag.at[li],
                device_id=(down[li],), device_id_type=pl.DeviceIdType.MESH)
            rdma.start()
            rdmas.append(rdma)
        for li in range(NLANES):
            rdmas[li].wait()
        for li in range(NLANES):
            pl.semaphore_signal(ack_ag.at[li], inc=1, device_id=(up[li],),
                                device_id_type=pl.DeviceIdType.MESH)
        for li in range(NLANES):
            pl.semaphore_wait(ack_ag.at[li], 1)


def _ring_allreduce(partial):
    return pl.pallas_call(
        _allreduce_body,
        out_shape=jax.ShapeDtypeStruct((M, N), jnp.float32),
        in_specs=[pl.BlockSpec(memory_space=pl.ANY)],
        out_specs=pl.BlockSpec(memory_space=pl.ANY),
        scratch_shapes=[
            pltpu.VMEM((NLANES, CHUNK_M, LANE_W), jnp.float32),
            pltpu.VMEM((NLANES, CHUNK_M, LANE_W), jnp.float32),
            pltpu.VMEM((NLANES, CHUNK_M, LANE_W), jnp.float32),
            pltpu.SemaphoreType.DMA((NLANES,)),
            pltpu.SemaphoreType.DMA((NLANES,)),
            pltpu.SemaphoreType.DMA((NLANES,)),
            pltpu.SemaphoreType.DMA((NLANES,)),
            pltpu.SemaphoreType.DMA((NLANES,)),
            pltpu.SemaphoreType.REGULAR((NLANES,)),
            pltpu.SemaphoreType.REGULAR((NLANES,)),
        ],
        compiler_params=pltpu.CompilerParams(collective_id=0),
    )(partial)


def kernel(x, w_mat, scale_x, scale_w):
    partial = jnp.dot(x.astype(jnp.bfloat16), w_mat.astype(jnp.bfloat16),
                      preferred_element_type=jnp.float32)
    partial = partial * (scale_x[0] * scale_w[0])
    return _ring_allreduce(partial)
